# baseline (speedup 1.0000x reference)
import functools

import jax
import jax.numpy as jnp
import numpy as np
from jax import lax
from jax.experimental import pallas as pl
from jax.experimental.pallas import tpu as pltpu
from jax.experimental.pallas import tpu_sc as plsc

NC, NS, LANES = 2, 16, 16
NW = NC * NS
B = 128
D = 256
NEG = -3.0e38

_MESH = plsc.VectorSubcoreMesh(core_axis_name="c", subcore_axis_name="s")
_SC_PARAMS = pltpu.CompilerParams(needs_layout_passes=False,
                                  use_tc_tiling_on_sc=False)

def _lane_iota():
    return lax.broadcasted_iota(jnp.int32, (LANES,), 0)


def _lane_gather(vec16, idx16):
    dn = lax.GatherDimensionNumbers(offset_dims=(), collapsed_slice_dims=(0,),
                                    start_index_map=(0,))
    return lax.gather(vec16, idx16.reshape(LANES, 1), dn, slice_sizes=(1,),
                      mode=lax.GatherScatterMode.PROMISE_IN_BOUNDS)


def _bcast_lane(vec16, lane, j):
    return _lane_gather(vec16, lane * 0 + j)


def _lane_sum(acc, lane):
    for sh in (1, 2, 4, 8):
        acc = acc + _lane_gather(acc, lane ^ sh)
    return acc


def _mm_body(x_ref, w_ref, b_ref, o_ref):
    o_ref[...] = (
        jnp.dot(x_ref[...], w_ref[...], preferred_element_type=jnp.float32)
        + b_ref[...]
    )


def _mm_res_body(x_ref, w_ref, b_ref, r_ref, o_ref):
    o_ref[...] = (
        jnp.dot(x_ref[...], w_ref[...], preferred_element_type=jnp.float32)
        + b_ref[...] + r_ref[...]
    )


def _matmul_bias(x, w, b, res=None, block_rows=1024):
    r, d = x.shape
    dout = w.shape[1]
    assert r % block_rows == 0, (r, block_rows)
    grid = (r // block_rows,)
    if res is None:
        return pl.pallas_call(
            _mm_body,
            grid=grid,
            in_specs=[
                pl.BlockSpec((block_rows, d), lambda i: (i, 0)),
                pl.BlockSpec((d, dout), lambda i: (0, 0)),
                pl.BlockSpec((1, dout), lambda i: (0, 0)),
            ],
            out_specs=pl.BlockSpec((block_rows, dout), lambda i: (i, 0)),
            out_shape=jax.ShapeDtypeStruct((r, dout), jnp.float32),
        )(x, w, b.reshape(1, dout))
    return pl.pallas_call(
        _mm_res_body,
        grid=grid,
        in_specs=[
            pl.BlockSpec((block_rows, d), lambda i: (i, 0)),
            pl.BlockSpec((d, dout), lambda i: (0, 0)),
            pl.BlockSpec((1, dout), lambda i: (0, 0)),
            pl.BlockSpec((block_rows, dout), lambda i: (i, 0)),
        ],
        out_specs=pl.BlockSpec((block_rows, dout), lambda i: (i, 0)),
        out_shape=jax.ShapeDtypeStruct((r, dout), jnp.float32),
    )(x, w, b.reshape(1, dout), res)


def _qkv(x, p, scale):
    w = jnp.concatenate([p['Wq'] * scale, p['Wk'], p['Wv']], axis=1)
    b = jnp.concatenate([p['bq'] * scale, p['bk'], p['bv']], axis=0)
    out = _matmul_bias(x, w, b)
    return out[:, :D], out[:, D:2 * D], out[:, 2 * D:]


@functools.lru_cache(maxsize=None)
def _k1_logits(EP, NdP, use_max):
    chunk = EP // NW
    nblk = chunk // B

    scratch = [
        pltpu.VMEM((B,), jnp.int32),
        pltpu.VMEM((B,), jnp.int32),
        pltpu.VMEM((B, D), jnp.float32),
        pltpu.VMEM((B, D), jnp.float32),
        pltpu.VMEM((B,), jnp.float32),
    ]
    if use_max:
        scratch.append(pltpu.VMEM((NdP,), jnp.float32))
        out_type = (jax.ShapeDtypeStruct((EP,), jnp.float32),
                    jax.ShapeDtypeStruct((NW, NdP), jnp.float32))
    else:
        out_type = jax.ShapeDtypeStruct((EP,), jnp.float32)

    @functools.partial(pl.kernel, out_type=out_type, mesh=_MESH,
                       scratch_types=scratch, compiler_params=_SC_PARAMS)
    def k1(q_hbm, k_hbm, src_hbm, dst_hbm, *rest):
        if use_max:
            l_hbm, mpart_hbm, src_v, dst_v, q_v, k_v, l_v, m_v = rest
        else:
            l_hbm, src_v, dst_v, q_v, k_v, l_v = rest
        w = lax.axis_index("s") * NC + lax.axis_index("c")
        base = w * chunk

        if use_max:
            def initm(i, carry):
                m_v[pl.ds(i * LANES, LANES)] = jnp.full((LANES,), NEG,
                                                        jnp.float32)
                return carry
            lax.fori_loop(0, NdP // LANES, initm, 0)

        def blk(b, carry):
            off = base + b * B
            pltpu.sync_copy(src_hbm.at[pl.ds(off, B)], src_v)
            pltpu.sync_copy(dst_hbm.at[pl.ds(off, B)], dst_v)
            pltpu.sync_copy(q_hbm.at[dst_v], q_v)
            pltpu.sync_copy(k_hbm.at[src_v], k_v)

            def dotgrp(g, ecarry):
                lane = _lane_iota()
                lvec = jnp.zeros((LANES,), jnp.float32)
                for j in range(LANES):
                    e = g * LANES + j
                    acc = q_v[e, pl.ds(0, LANES)] * k_v[e, pl.ds(0, LANES)]
                    for c in range(1, D // LANES):
                        acc = acc + (q_v[e, pl.ds(c * LANES, LANES)]
                                     * k_v[e, pl.ds(c * LANES, LANES)])
                    lvec = jnp.where(lane == j, _lane_sum(acc, lane), lvec)
                l_v[pl.ds(g * LANES, LANES)] = lvec
                return ecarry
            lax.fori_loop(0, B // LANES, dotgrp, 0)

            if use_max:
                def grp(g, gcarry):
                    sl = pl.ds(g * LANES, LANES)
                    l16 = l_v[sl]
                    d16 = dst_v[sl]

                    def cond(c_):
                        return c_

                    def body(c_):
                        mo = plsc.load_gather(m_v, [d16])
                        plsc.store_scatter(m_v, [d16], l16, mask=l16 > mo)
                        mo2 = plsc.load_gather(m_v, [d16])
                        return jnp.any(l16 > mo2)
                    lax.while_loop(cond, body, True)
                    return gcarry
                lax.fori_loop(0, B // LANES, grp, 0)

            pltpu.sync_copy(l_v, l_hbm.at[pl.ds(off, B)])
            return carry
        lax.fori_loop(0, nblk, blk, 0)

        if use_max:
            pltpu.sync_copy(m_v, mpart_hbm.at[w])
    return k1


@functools.lru_cache(maxsize=None)
def _k1_fused(EP, NdP):
    chunk = EP // NW
    nblk = chunk // B
    sl16 = NdP // NS

    scratch = [
        pltpu.VMEM((B,), jnp.int32),
        pltpu.VMEM((B,), jnp.int32),
        pltpu.VMEM((B, D), jnp.float32),
        pltpu.VMEM((B, D), jnp.float32),
        pltpu.VMEM((B,), jnp.float32),
        pltpu.VMEM((sl16,), jnp.float32),
        pltpu.VMEM_SHARED((NdP,), jnp.float32),
    ]
    out_type = (jax.ShapeDtypeStruct((EP,), jnp.float32),
                jax.ShapeDtypeStruct((NC, NdP), jnp.float32))

    @functools.partial(pl.kernel, out_type=out_type, mesh=_MESH,
                       scratch_types=scratch, compiler_params=_SC_PARAMS)
    def k1e(q_hbm, k_hbm, src_hbm, dst_hbm, e_hbm, spart_hbm,
            src_v, dst_v, q_v, k_v, e_v, z_v, s_sh):
        c = lax.axis_index("c")
        s = lax.axis_index("s")
        w = s * NC + c
        base = w * chunk

        def zb(i, carry):
            z_v[pl.ds(i * LANES, LANES)] = jnp.zeros((LANES,), jnp.float32)
            return carry
        lax.fori_loop(0, sl16 // LANES, zb, 0)
        pltpu.sync_copy(z_v, s_sh.at[pl.ds(s * sl16, sl16)])
        plsc.subcore_barrier()

        def blk(b, carry):
            off = base + b * B
            pltpu.sync_copy(src_hbm.at[pl.ds(off, B)], src_v)
            pltpu.sync_copy(dst_hbm.at[pl.ds(off, B)], dst_v)
            pltpu.sync_copy(q_hbm.at[dst_v], q_v)
            pltpu.sync_copy(k_hbm.at[src_v], k_v)

            def dotgrp(g, ecarry):
                lane = _lane_iota()
                lvec = jnp.zeros((LANES,), jnp.float32)
                for j in range(LANES):
                    e = g * LANES + j
                    acc = q_v[e, pl.ds(0, LANES)] * k_v[e, pl.ds(0, LANES)]
                    for cc in range(1, D // LANES):
                        acc = acc + (q_v[e, pl.ds(cc * LANES, LANES)]
                                     * k_v[e, pl.ds(cc * LANES, LANES)])
                    lvec = jnp.where(lane == j, _lane_sum(acc, lane), lvec)
                e_v[pl.ds(g * LANES, LANES)] = jnp.exp(lvec)
                return ecarry
            lax.fori_loop(0, B // LANES, dotgrp, 0)

            pltpu.sync_copy(e_v, e_hbm.at[pl.ds(off, B)])
            pltpu.sync_copy(e_v, s_sh.at[dst_v], add=True)
            return carry
        lax.fori_loop(0, nblk, blk, 0)

        plsc.subcore_barrier()
        pltpu.sync_copy(s_sh.at[pl.ds(s * sl16, sl16)],
                        spart_hbm.at[c, pl.ds(s * sl16, sl16)])
    return k1e


@functools.lru_cache(maxsize=None)
def _k2_merge(NdP):
    sl_len = NdP // NW

    @functools.partial(
        pl.kernel,
        out_type=jax.ShapeDtypeStruct((NdP,), jnp.float32),
        mesh=_MESH,
        scratch_types=[pltpu.VMEM((sl_len,), jnp.float32),
                       pltpu.VMEM((sl_len,), jnp.float32)],
        compiler_params=_SC_PARAMS)
    def k2(mpart_hbm, mfin_hbm, acc_v, tmp_v):
        w = lax.axis_index("s") * NC + lax.axis_index("c")
        off = w * sl_len
        pltpu.sync_copy(mpart_hbm.at[0, pl.ds(off, sl_len)], acc_v)

        def red(w2, carry):
            pltpu.sync_copy(mpart_hbm.at[w2, pl.ds(off, sl_len)], tmp_v)

            def ch(i, icarry):
                s_ = pl.ds(i * LANES, LANES)
                acc_v[s_] = jnp.maximum(acc_v[s_], tmp_v[s_])
                return icarry
            lax.fori_loop(0, sl_len // LANES, ch, 0)
            return carry
        lax.fori_loop(1, NW, red, 0)
        pltpu.sync_copy(acc_v, mfin_hbm.at[pl.ds(off, sl_len)])
    return k2


@functools.lru_cache(maxsize=None)
def _k3_expsum(EP, NdP, use_max):
    chunk = EP // NW
    nblk = chunk // B
    sl16 = NdP // NS

    scratch = [
        pltpu.VMEM((B,), jnp.int32),
        pltpu.VMEM((B,), jnp.float32),
        pltpu.VMEM((B,), jnp.float32),
        pltpu.VMEM((sl16,), jnp.float32),
        pltpu.VMEM_SHARED((NdP,), jnp.float32),
    ]
    if use_max:
        scratch.append(pltpu.VMEM((NdP,), jnp.float32))

    out_type = (jax.ShapeDtypeStruct((EP,), jnp.float32),
                jax.ShapeDtypeStruct((NC, NdP), jnp.float32))

    @functools.partial(pl.kernel, out_type=out_type, mesh=_MESH,
                       scratch_types=scratch, compiler_params=_SC_PARAMS)
    def k3(dst_hbm, l_hbm, *rest):
        if use_max:
            mfin_hbm, e_hbm, spart_hbm, dst_v, l_v, e_v, z_v, s_sh, m_v = rest
        else:
            e_hbm, spart_hbm, dst_v, l_v, e_v, z_v, s_sh = rest
        c = lax.axis_index("c")
        s = lax.axis_index("s")
        w = s * NC + c

        def zb(i, carry):
            z_v[pl.ds(i * LANES, LANES)] = jnp.zeros((LANES,), jnp.float32)
            return carry
        lax.fori_loop(0, sl16 // LANES, zb, 0)
        pltpu.sync_copy(z_v, s_sh.at[pl.ds(s * sl16, sl16)])
        if use_max:
            pltpu.sync_copy(mfin_hbm, m_v)
        plsc.subcore_barrier()

        def blk(b, carry):
            off = w * chunk + b * B
            pltpu.sync_copy(dst_hbm.at[pl.ds(off, B)], dst_v)
            pltpu.sync_copy(l_hbm.at[pl.ds(off, B)], l_v)

            def grp(g, gcarry):
                sl = pl.ds(g * LANES, LANES)
                l16 = l_v[sl]
                if use_max:
                    m16 = plsc.load_gather(m_v, [dst_v[sl]])
                    e_v[sl] = jnp.exp(l16 - m16)
                else:
                    e_v[sl] = jnp.exp(l16)
                return gcarry
            lax.fori_loop(0, B // LANES, grp, 0)
            pltpu.sync_copy(e_v, e_hbm.at[pl.ds(off, B)])
            pltpu.sync_copy(e_v, s_sh.at[dst_v], add=True)
            return carry
        lax.fori_loop(0, nblk, blk, 0)

        plsc.subcore_barrier()
        pltpu.sync_copy(s_sh.at[pl.ds(s * sl16, sl16)],
                        spart_hbm.at[c, pl.ds(s * sl16, sl16)])
    return k3


@functools.lru_cache(maxsize=None)
def _k5_scatter(EP, NdP, DSUB):
    npass = D // DSUB
    chunk = EP // NW
    nblk = chunk // B
    rows16 = NdP // NS

    scratch = [
        pltpu.VMEM((B,), jnp.int32),
        pltpu.VMEM((B,), jnp.int32),
        pltpu.VMEM((B,), jnp.float32),
        pltpu.VMEM((B, DSUB), jnp.float32),
        pltpu.VMEM((B, DSUB), jnp.float32),
        pltpu.VMEM((NdP,), jnp.float32),
        pltpu.VMEM_SHARED((NdP, DSUB), jnp.float32),
    ]

    @functools.partial(
        pl.kernel,
        out_type=jax.ShapeDtypeStruct((npass * NC, NdP, DSUB), jnp.float32),
        mesh=_MESH, scratch_types=scratch, compiler_params=_SC_PARAMS)
    def k5(*args):
        vq = args[:npass]
        (src_hbm, dst_hbm, e_hbm, s_hbm, out_hbm,
         src_v, dst_v, e_v, vr_v, sc_v, s_v, slab) = args[npass:]
        c = lax.axis_index("c")
        s = lax.axis_index("s")
        w = s * NC + c

        pltpu.sync_copy(s_hbm, s_v)

        def zrow(i, carry):
            for c16 in range(DSUB // LANES):
                sc_v[i, pl.ds(c16 * LANES, LANES)] = jnp.zeros(
                    (LANES,), jnp.float32)
            return carry

        for p in range(npass):
            lax.fori_loop(0, B, zrow, 0)
            for q0 in range(0, rows16, B):
                n = min(B, rows16 - q0)
                pltpu.sync_copy(sc_v.at[pl.ds(0, n)],
                                slab.at[pl.ds(s * rows16 + q0, n)])
            plsc.subcore_barrier()

            def blk(b, carry):
                off = w * chunk + b * B
                pltpu.sync_copy(src_hbm.at[pl.ds(off, B)], src_v)
                pltpu.sync_copy(dst_hbm.at[pl.ds(off, B)], dst_v)
                pltpu.sync_copy(e_hbm.at[pl.ds(off, B)], e_v)
                pltpu.sync_copy(vq[p].at[src_v], vr_v)

                def grp(g, gcarry):
                    lane = _lane_iota()
                    sl = pl.ds(g * LANES, LANES)
                    d16 = dst_v[sl]
                    sv = plsc.load_gather(s_v, [d16])
                    a16 = e_v[sl] / (sv + 1e-16)
                    for j in range(LANES):
                        e = g * LANES + j
                        va = _bcast_lane(a16, lane, j)
                        for c16 in range(DSUB // LANES):
                            slc = pl.ds(c16 * LANES, LANES)
                            sc_v[e, slc] = vr_v[e, slc] * va
                    return gcarry
                lax.fori_loop(0, B // LANES, grp, 0)

                pltpu.sync_copy(sc_v, slab.at[dst_v], add=True)
                return carry
            lax.fori_loop(0, nblk, blk, 0)

            plsc.subcore_barrier()
            out_row = p * NC + c
            for q0 in range(0, rows16, B):
                n = min(B, rows16 - q0)
                pltpu.sync_copy(slab.at[pl.ds(s * rows16 + q0, n)],
                                out_hbm.at[out_row,
                                           pl.ds(s * rows16 + q0, n)])
            plsc.subcore_barrier()
    return k5


def _add_body(a_ref, b_ref, o_ref):
    o_ref[...] = a_ref[...] + b_ref[...]


def _pallas_add(a, b, block_rows=1024):
    r, d = a.shape
    grid = (r // block_rows,)
    return pl.pallas_call(
        _add_body,
        grid=grid,
        in_specs=[pl.BlockSpec((block_rows, d), lambda i: (i, 0)),
                  pl.BlockSpec((block_rows, d), lambda i: (i, 0))],
        out_specs=pl.BlockSpec((block_rows, d), lambda i: (i, 0)),
        out_shape=jax.ShapeDtypeStruct((r, d), jnp.float32),
    )(a, b)


def _round_up(x, m):
    return (x + m - 1) // m * m


def _sparse_attn_sc(q, k, v, src, dst, use_max):
    NdP = q.shape[0]
    EP = src.shape[0]
    if use_max:
        l, mpart = _k1_logits(EP, NdP, True)(q, k, src, dst)
        mfin = _k2_merge(NdP)(mpart)
        e, spart = _k3_expsum(EP, NdP, True)(dst, l, mfin)
    else:
        e, spart = _k1_fused(EP, NdP)(q, k, src, dst)
    dsub = 128 if NdP <= 12288 else 64
    npass = D // dsub
    vq = [v[:, i * dsub:(i + 1) * dsub] for i in range(npass)]
    srows = NdP // 128
    s = _pallas_add(spart[0].reshape(srows, 128),
                    spart[1].reshape(srows, 128),
                    block_rows=srows).reshape(NdP)
    outp = _k5_scatter(EP, NdP, dsub)(*vq, src, dst, e, s)
    a = jnp.transpose(outp[0::NC], (1, 0, 2)).reshape(NdP, D)
    b = jnp.transpose(outp[1::NC], (1, 0, 2)).reshape(NdP, D)
    return _pallas_add(a, b)


def _pad_edges(src, dst, ns, nd):
    E = src.shape[0]
    EP = _round_up(E, NW * B)
    src = jnp.pad(src.astype(jnp.int32), (0, EP - E), constant_values=ns)
    dst = jnp.pad(dst.astype(jnp.int32), (0, EP - E), constant_values=nd)
    return src, dst


def kernel(x_node, x_tri, params, node_edge_index, tri_edge_index,
           nt_edge_index, tn_edge_index):
    t, n, d = x_node.shape
    _, m, _ = x_tri.shape
    assert t == 1 and d == D
    NP = _round_up(n, 1024)
    MP = _round_up(m, 1024)
    scale = d ** (-0.5)

    xn = jnp.pad(x_node.reshape(n, d), ((0, NP - n), (0, 0)))
    xt = jnp.pad(x_tri.reshape(m, d), ((0, MP - m), (0, 0)))

    ne_src, ne_dst = _pad_edges(node_edge_index[0], node_edge_index[1], n, n)
    te_src, te_dst = _pad_edges(tri_edge_index[0], tri_edge_index[1], m, m)
    nt_src, nt_dst = _pad_edges(nt_edge_index[0], nt_edge_index[1], n, m)
    tn_src, tn_dst = _pad_edges(tn_edge_index[0], tn_edge_index[1], m, n)

    p = params['nsa']
    q, k, v = _qkv(xn, p, scale)
    h_node = _sparse_attn_sc(q, k, v, ne_src, ne_dst, False)

    p = params['n2t']
    q = _matmul_bias(xt, p['Wq'] * scale, p['bq'] * scale)
    kv = _matmul_bias(h_node,
                      jnp.concatenate([p['Wk'], p['Wv']], axis=1),
                      jnp.concatenate([p['bk'], p['bv']], axis=0))
    aggr = _sparse_attn_sc(q, kv[:, :D], kv[:, D:], nt_src, nt_dst, False)
    h_tri = _matmul_bias(aggr, p['Wo'], p['bo'], res=xt)

    p = params['tsa']
    q, k, v = _qkv(h_tri, p, d ** 0.5)
    h_tri2 = _sparse_attn_sc(q, k, v, te_src, te_dst, True)

    p = params['t2n']
    q = _matmul_bias(h_node, p['Wq'] * scale, p['bq'] * scale)
    kv = _matmul_bias(h_tri2,
                      jnp.concatenate([p['Wk'], p['Wv']], axis=1),
                      jnp.concatenate([p['bk'], p['bv']], axis=0))
    aggr = _sparse_attn_sc(q, kv[:, :D], kv[:, D:], tn_src, tn_dst, False)
    out_node = _matmul_bias(aggr, p['Wo'], p['bo'], res=h_node)
    return out_node[:n].reshape(t, n, d)


# speedup vs baseline: 1.1333x; 1.1333x over previous
import functools

import jax
import jax.numpy as jnp
import numpy as np
from jax import lax
from jax.experimental import pallas as pl
from jax.experimental.pallas import tpu as pltpu
from jax.experimental.pallas import tpu_sc as plsc

NC, NS, LANES = 2, 16, 16
NW = NC * NS
B = 128
D = 256
NEG = -3.0e38

_MESH = plsc.VectorSubcoreMesh(core_axis_name="c", subcore_axis_name="s")
_SC_PARAMS = pltpu.CompilerParams(needs_layout_passes=False,
                                  use_tc_tiling_on_sc=False)

def _lane_iota():
    return lax.broadcasted_iota(jnp.int32, (LANES,), 0)


def _lane_gather(vec16, idx16):
    dn = lax.GatherDimensionNumbers(offset_dims=(), collapsed_slice_dims=(0,),
                                    start_index_map=(0,))
    return lax.gather(vec16, idx16.reshape(LANES, 1), dn, slice_sizes=(1,),
                      mode=lax.GatherScatterMode.PROMISE_IN_BOUNDS)


def _bcast_lane(vec16, lane, j):
    return _lane_gather(vec16, lane * 0 + j)


def _lane_sum(acc, lane):
    for sh in (1, 2, 4, 8):
        acc = acc + _lane_gather(acc, lane ^ sh)
    return acc


def _mm_body(x_ref, w_ref, b_ref, o_ref):
    o_ref[...] = (
        jnp.dot(x_ref[...], w_ref[...], preferred_element_type=jnp.float32)
        + b_ref[...]
    )


def _mm_res_body(x_ref, w_ref, b_ref, r_ref, o_ref):
    o_ref[...] = (
        jnp.dot(x_ref[...], w_ref[...], preferred_element_type=jnp.float32)
        + b_ref[...] + r_ref[...]
    )


def _matmul_bias(x, w, b, res=None, block_rows=1024):
    r, d = x.shape
    dout = w.shape[1]
    assert r % block_rows == 0, (r, block_rows)
    grid = (r // block_rows,)
    if res is None:
        return pl.pallas_call(
            _mm_body,
            grid=grid,
            in_specs=[
                pl.BlockSpec((block_rows, d), lambda i: (i, 0)),
                pl.BlockSpec((d, dout), lambda i: (0, 0)),
                pl.BlockSpec((1, dout), lambda i: (0, 0)),
            ],
            out_specs=pl.BlockSpec((block_rows, dout), lambda i: (i, 0)),
            out_shape=jax.ShapeDtypeStruct((r, dout), jnp.float32),
        )(x, w, b.reshape(1, dout))
    return pl.pallas_call(
        _mm_res_body,
        grid=grid,
        in_specs=[
            pl.BlockSpec((block_rows, d), lambda i: (i, 0)),
            pl.BlockSpec((d, dout), lambda i: (0, 0)),
            pl.BlockSpec((1, dout), lambda i: (0, 0)),
            pl.BlockSpec((block_rows, dout), lambda i: (i, 0)),
        ],
        out_specs=pl.BlockSpec((block_rows, dout), lambda i: (i, 0)),
        out_shape=jax.ShapeDtypeStruct((r, dout), jnp.float32),
    )(x, w, b.reshape(1, dout), res)


def _qkv(x, p, scale):
    w = jnp.concatenate([p['Wq'] * scale, p['Wk'], p['Wv']], axis=1)
    b = jnp.concatenate([p['bq'] * scale, p['bk'], p['bv']], axis=0)
    out = _matmul_bias(x, w, b)
    return out[:, :D], out[:, D:2 * D], out[:, 2 * D:]


@functools.lru_cache(maxsize=None)
def _k1_logits(EP, NdP, use_max):
    chunk = EP // NW
    nblk = chunk // B

    scratch = [
        pltpu.VMEM((B,), jnp.int32),
        pltpu.VMEM((B,), jnp.int32),
        pltpu.VMEM((B, D), jnp.float32),
        pltpu.VMEM((B, D), jnp.float32),
        pltpu.VMEM((B,), jnp.float32),
    ]
    if use_max:
        scratch.append(pltpu.VMEM((NdP,), jnp.float32))
        out_type = (jax.ShapeDtypeStruct((EP,), jnp.float32),
                    jax.ShapeDtypeStruct((NW, NdP), jnp.float32))
    else:
        out_type = jax.ShapeDtypeStruct((EP,), jnp.float32)

    @functools.partial(pl.kernel, out_type=out_type, mesh=_MESH,
                       scratch_types=scratch, compiler_params=_SC_PARAMS)
    def k1(q_hbm, k_hbm, src_hbm, dst_hbm, *rest):
        if use_max:
            l_hbm, mpart_hbm, src_v, dst_v, q_v, k_v, l_v, m_v = rest
        else:
            l_hbm, src_v, dst_v, q_v, k_v, l_v = rest
        w = lax.axis_index("s") * NC + lax.axis_index("c")
        base = w * chunk

        if use_max:
            def initm(i, carry):
                m_v[pl.ds(i * LANES, LANES)] = jnp.full((LANES,), NEG,
                                                        jnp.float32)
                return carry
            lax.fori_loop(0, NdP // LANES, initm, 0)

        def blk(b, carry):
            off = base + b * B
            pltpu.sync_copy(src_hbm.at[pl.ds(off, B)], src_v)
            pltpu.sync_copy(dst_hbm.at[pl.ds(off, B)], dst_v)
            pltpu.sync_copy(q_hbm.at[dst_v], q_v)
            pltpu.sync_copy(k_hbm.at[src_v], k_v)

            def dotgrp(g, ecarry):
                lane = _lane_iota()
                lvec = jnp.zeros((LANES,), jnp.float32)
                for j in range(LANES):
                    e = g * LANES + j
                    acc = q_v[e, pl.ds(0, LANES)] * k_v[e, pl.ds(0, LANES)]
                    for c in range(1, D // LANES):
                        acc = acc + (q_v[e, pl.ds(c * LANES, LANES)]
                                     * k_v[e, pl.ds(c * LANES, LANES)])
                    lvec = jnp.where(lane == j, _lane_sum(acc, lane), lvec)
                l_v[pl.ds(g * LANES, LANES)] = lvec
                return ecarry
            lax.fori_loop(0, B // LANES, dotgrp, 0)

            if use_max:
                def grp(g, gcarry):
                    sl = pl.ds(g * LANES, LANES)
                    l16 = l_v[sl]
                    d16 = dst_v[sl]

                    def cond(c_):
                        return c_

                    def body(c_):
                        mo = plsc.load_gather(m_v, [d16])
                        plsc.store_scatter(m_v, [d16], l16, mask=l16 > mo)
                        mo2 = plsc.load_gather(m_v, [d16])
                        return jnp.any(l16 > mo2)
                    lax.while_loop(cond, body, True)
                    return gcarry
                lax.fori_loop(0, B // LANES, grp, 0)

            pltpu.sync_copy(l_v, l_hbm.at[pl.ds(off, B)])
            return carry
        lax.fori_loop(0, nblk, blk, 0)

        if use_max:
            pltpu.sync_copy(m_v, mpart_hbm.at[w])
    return k1


BE = 64


@functools.lru_cache(maxsize=None)
def _k1_fused(EP, NdP):
    chunk = EP // NW
    nblk = chunk // BE
    sl16 = NdP // NS
    assert nblk % 2 == 0

    scratch = [
        pltpu.VMEM((2 * BE,), jnp.int32),
        pltpu.VMEM((2 * BE,), jnp.int32),
        pltpu.VMEM((2 * BE, D), jnp.float32),
        pltpu.VMEM((2 * BE, D), jnp.float32),
        pltpu.VMEM((BE,), jnp.int32),
        pltpu.VMEM((BE,), jnp.float32),
        pltpu.VMEM((sl16,), jnp.float32),
        pltpu.VMEM_SHARED((NdP,), jnp.float32),
        pltpu.SemaphoreType.DMA,
    ]
    out_type = (jax.ShapeDtypeStruct((EP,), jnp.float32),
                jax.ShapeDtypeStruct((NC, NdP), jnp.float32))

    @functools.partial(pl.kernel, out_type=out_type, mesh=_MESH,
                       scratch_types=scratch, compiler_params=_SC_PARAMS)
    def k1e(qk_hbm, sd_hbm, dst_hbm, e_hbm, spart_hbm,
            i0, i1, qk0, qk1, dst_v, e_v, z_v, s_sh, gsem):
        c = lax.axis_index("c")
        s = lax.axis_index("s")
        w = s * NC + c
        base = w * chunk
        gblk = base // BE
        ibufs = (i0, i1)
        qkbufs = (qk0, qk1)

        def zb(i, carry):
            z_v[pl.ds(i * LANES, LANES)] = jnp.zeros((LANES,), jnp.float32)
            return carry
        lax.fori_loop(0, sl16 // LANES, zb, 0)
        pltpu.sync_copy(z_v, s_sh.at[pl.ds(s * sl16, sl16)])
        plsc.subcore_barrier()

        pltpu.sync_copy(sd_hbm.at[gblk], i0)
        pltpu.async_copy(qk_hbm.at[i0], qk0, gsem)

        def pair(g, carry):
            for b01 in range(2):
                t = g * 2 + b01
                icur, qkc = ibufs[b01], qkbufs[b01]
                inxt, qkn = ibufs[1 - b01], qkbufs[1 - b01]
                pltpu.make_async_copy(qk_hbm.at[pl.ds(0, 2 * BE)], qkc,
                                      gsem).wait()

                @pl.when(t + 1 < nblk)
                def _():
                    pltpu.sync_copy(sd_hbm.at[gblk + t + 1], inxt)
                    pltpu.async_copy(qk_hbm.at[inxt], qkn, gsem)

                off = base + t * BE
                pltpu.sync_copy(dst_hbm.at[pl.ds(off, BE)], dst_v)

                def dotgrp(gg, ecarry):
                    lane = _lane_iota()
                    lvec = jnp.zeros((LANES,), jnp.float32)
                    for j in range(LANES):
                        e = gg * LANES + j
                        acc = (qkc[e, pl.ds(0, LANES)]
                               * qkc[BE + e, pl.ds(0, LANES)])
                        for cc in range(1, D // LANES):
                            acc = acc + (qkc[e, pl.ds(cc * LANES, LANES)]
                                         * qkc[BE + e,
                                               pl.ds(cc * LANES, LANES)])
                        lvec = jnp.where(lane == j, _lane_sum(acc, lane),
                                         lvec)
                    e_v[pl.ds(gg * LANES, LANES)] = jnp.exp(lvec)
                    return ecarry
                lax.fori_loop(0, BE // LANES, dotgrp, 0)

                pltpu.sync_copy(e_v, e_hbm.at[pl.ds(off, BE)])
                pltpu.sync_copy(e_v, s_sh.at[dst_v], add=True)
            return carry
        lax.fori_loop(0, nblk // 2, pair, 0)

        plsc.subcore_barrier()
        pltpu.sync_copy(s_sh.at[pl.ds(s * sl16, sl16)],
                        spart_hbm.at[c, pl.ds(s * sl16, sl16)])
    return k1e


@functools.lru_cache(maxsize=None)
def _k2_merge(NdP):
    sl_len = NdP // NW

    @functools.partial(
        pl.kernel,
        out_type=jax.ShapeDtypeStruct((NdP,), jnp.float32),
        mesh=_MESH,
        scratch_types=[pltpu.VMEM((sl_len,), jnp.float32),
                       pltpu.VMEM((sl_len,), jnp.float32)],
        compiler_params=_SC_PARAMS)
    def k2(mpart_hbm, mfin_hbm, acc_v, tmp_v):
        w = lax.axis_index("s") * NC + lax.axis_index("c")
        off = w * sl_len
        pltpu.sync_copy(mpart_hbm.at[0, pl.ds(off, sl_len)], acc_v)

        def red(w2, carry):
            pltpu.sync_copy(mpart_hbm.at[w2, pl.ds(off, sl_len)], tmp_v)

            def ch(i, icarry):
                s_ = pl.ds(i * LANES, LANES)
                acc_v[s_] = jnp.maximum(acc_v[s_], tmp_v[s_])
                return icarry
            lax.fori_loop(0, sl_len // LANES, ch, 0)
            return carry
        lax.fori_loop(1, NW, red, 0)
        pltpu.sync_copy(acc_v, mfin_hbm.at[pl.ds(off, sl_len)])
    return k2


@functools.lru_cache(maxsize=None)
def _k3_expsum(EP, NdP, use_max):
    chunk = EP // NW
    nblk = chunk // B
    sl16 = NdP // NS

    scratch = [
        pltpu.VMEM((B,), jnp.int32),
        pltpu.VMEM((B,), jnp.float32),
        pltpu.VMEM((B,), jnp.float32),
        pltpu.VMEM((sl16,), jnp.float32),
        pltpu.VMEM_SHARED((NdP,), jnp.float32),
    ]
    if use_max:
        scratch.append(pltpu.VMEM((NdP,), jnp.float32))

    out_type = (jax.ShapeDtypeStruct((EP,), jnp.float32),
                jax.ShapeDtypeStruct((NC, NdP), jnp.float32))

    @functools.partial(pl.kernel, out_type=out_type, mesh=_MESH,
                       scratch_types=scratch, compiler_params=_SC_PARAMS)
    def k3(dst_hbm, l_hbm, *rest):
        if use_max:
            mfin_hbm, e_hbm, spart_hbm, dst_v, l_v, e_v, z_v, s_sh, m_v = rest
        else:
            e_hbm, spart_hbm, dst_v, l_v, e_v, z_v, s_sh = rest
        c = lax.axis_index("c")
        s = lax.axis_index("s")
        w = s * NC + c

        def zb(i, carry):
            z_v[pl.ds(i * LANES, LANES)] = jnp.zeros((LANES,), jnp.float32)
            return carry
        lax.fori_loop(0, sl16 // LANES, zb, 0)
        pltpu.sync_copy(z_v, s_sh.at[pl.ds(s * sl16, sl16)])
        if use_max:
            pltpu.sync_copy(mfin_hbm, m_v)
        plsc.subcore_barrier()

        def blk(b, carry):
            off = w * chunk + b * B
            pltpu.sync_copy(dst_hbm.at[pl.ds(off, B)], dst_v)
            pltpu.sync_copy(l_hbm.at[pl.ds(off, B)], l_v)

            def grp(g, gcarry):
                sl = pl.ds(g * LANES, LANES)
                l16 = l_v[sl]
                if use_max:
                    m16 = plsc.load_gather(m_v, [dst_v[sl]])
                    e_v[sl] = jnp.exp(l16 - m16)
                else:
                    e_v[sl] = jnp.exp(l16)
                return gcarry
            lax.fori_loop(0, B // LANES, grp, 0)
            pltpu.sync_copy(e_v, e_hbm.at[pl.ds(off, B)])
            pltpu.sync_copy(e_v, s_sh.at[dst_v], add=True)
            return carry
        lax.fori_loop(0, nblk, blk, 0)

        plsc.subcore_barrier()
        pltpu.sync_copy(s_sh.at[pl.ds(s * sl16, sl16)],
                        spart_hbm.at[c, pl.ds(s * sl16, sl16)])
    return k3


@functools.lru_cache(maxsize=None)
def _k5_scatter(EP, NdP, DSUB):
    npass = D // DSUB
    chunk = EP // NW
    nblk = chunk // B
    rows16 = NdP // NS

    scratch = [
        pltpu.VMEM((B,), jnp.int32),
        pltpu.VMEM((B,), jnp.int32),
        pltpu.VMEM((B,), jnp.float32),
        pltpu.VMEM((B, DSUB), jnp.float32),
        pltpu.VMEM((B, DSUB), jnp.float32),
        pltpu.VMEM((NdP,), jnp.float32),
        pltpu.VMEM_SHARED((NdP, DSUB), jnp.float32),
    ]

    @functools.partial(
        pl.kernel,
        out_type=jax.ShapeDtypeStruct((npass * NC, NdP, DSUB), jnp.float32),
        mesh=_MESH, scratch_types=scratch, compiler_params=_SC_PARAMS)
    def k5(*args):
        vq = args[:npass]
        (src_hbm, dst_hbm, e_hbm, s_hbm, out_hbm,
         src_v, dst_v, e_v, vr_v, sc_v, s_v, slab) = args[npass:]
        c = lax.axis_index("c")
        s = lax.axis_index("s")
        w = s * NC + c

        pltpu.sync_copy(s_hbm, s_v)

        def zrow(i, carry):
            for c16 in range(DSUB // LANES):
                sc_v[i, pl.ds(c16 * LANES, LANES)] = jnp.zeros(
                    (LANES,), jnp.float32)
            return carry

        for p in range(npass):
            lax.fori_loop(0, B, zrow, 0)
            for q0 in range(0, rows16, B):
                n = min(B, rows16 - q0)
                pltpu.sync_copy(sc_v.at[pl.ds(0, n)],
                                slab.at[pl.ds(s * rows16 + q0, n)])
            plsc.subcore_barrier()

            def blk(b, carry):
                off = w * chunk + b * B
                pltpu.sync_copy(src_hbm.at[pl.ds(off, B)], src_v)
                pltpu.sync_copy(dst_hbm.at[pl.ds(off, B)], dst_v)
                pltpu.sync_copy(e_hbm.at[pl.ds(off, B)], e_v)
                pltpu.sync_copy(vq[p].at[src_v], vr_v)

                def grp(g, gcarry):
                    lane = _lane_iota()
                    sl = pl.ds(g * LANES, LANES)
                    d16 = dst_v[sl]
                    sv = plsc.load_gather(s_v, [d16])
                    a16 = e_v[sl] / (sv + 1e-16)
                    for j in range(LANES):
                        e = g * LANES + j
                        va = _bcast_lane(a16, lane, j)
                        for c16 in range(DSUB // LANES):
                            slc = pl.ds(c16 * LANES, LANES)
                            sc_v[e, slc] = vr_v[e, slc] * va
                    return gcarry
                lax.fori_loop(0, B // LANES, grp, 0)

                pltpu.sync_copy(sc_v, slab.at[dst_v], add=True)
                return carry
            lax.fori_loop(0, nblk, blk, 0)

            plsc.subcore_barrier()
            out_row = p * NC + c
            for q0 in range(0, rows16, B):
                n = min(B, rows16 - q0)
                pltpu.sync_copy(slab.at[pl.ds(s * rows16 + q0, n)],
                                out_hbm.at[out_row,
                                           pl.ds(s * rows16 + q0, n)])
            plsc.subcore_barrier()
    return k5


def _add_body(a_ref, b_ref, o_ref):
    o_ref[...] = a_ref[...] + b_ref[...]


def _pallas_add(a, b, block_rows=1024):
    r, d = a.shape
    grid = (r // block_rows,)
    return pl.pallas_call(
        _add_body,
        grid=grid,
        in_specs=[pl.BlockSpec((block_rows, d), lambda i: (i, 0)),
                  pl.BlockSpec((block_rows, d), lambda i: (i, 0))],
        out_specs=pl.BlockSpec((block_rows, d), lambda i: (i, 0)),
        out_shape=jax.ShapeDtypeStruct((r, d), jnp.float32),
    )(a, b)


def _round_up(x, m):
    return (x + m - 1) // m * m


def _sparse_attn_sc(q, k, v, src, dst, use_max):
    NdP = q.shape[0]
    EP = src.shape[0]
    if use_max:
        l, mpart = _k1_logits(EP, NdP, True)(q, k, src, dst)
        mfin = _k2_merge(NdP)(mpart)
        e, spart = _k3_expsum(EP, NdP, True)(dst, l, mfin)
    else:
        qk = jnp.concatenate([q, k], axis=0)
        sd = jnp.concatenate([dst.reshape(-1, BE),
                              src.reshape(-1, BE) + NdP], axis=1)
        e, spart = _k1_fused(EP, NdP)(qk, sd, dst)
    dsub = 128 if NdP <= 12288 else 64
    npass = D // dsub
    vq = [v[:, i * dsub:(i + 1) * dsub] for i in range(npass)]
    srows = NdP // 128
    s = _pallas_add(spart[0].reshape(srows, 128),
                    spart[1].reshape(srows, 128),
                    block_rows=srows).reshape(NdP)
    outp = _k5_scatter(EP, NdP, dsub)(*vq, src, dst, e, s)
    a = jnp.transpose(outp[0::NC], (1, 0, 2)).reshape(NdP, D)
    b = jnp.transpose(outp[1::NC], (1, 0, 2)).reshape(NdP, D)
    return _pallas_add(a, b)


def _pad_edges(src, dst, ns, nd):
    E = src.shape[0]
    EP = _round_up(E, NW * B)
    src = jnp.pad(src.astype(jnp.int32), (0, EP - E), constant_values=ns)
    dst = jnp.pad(dst.astype(jnp.int32), (0, EP - E), constant_values=nd)
    return src, dst


def kernel(x_node, x_tri, params, node_edge_index, tri_edge_index,
           nt_edge_index, tn_edge_index):
    t, n, d = x_node.shape
    _, m, _ = x_tri.shape
    assert t == 1 and d == D
    NP = _round_up(n, 1024)
    MP = _round_up(m, 1024)
    scale = d ** (-0.5)

    xn = jnp.pad(x_node.reshape(n, d), ((0, NP - n), (0, 0)))
    xt = jnp.pad(x_tri.reshape(m, d), ((0, MP - m), (0, 0)))

    ne_src, ne_dst = _pad_edges(node_edge_index[0], node_edge_index[1], n, n)
    te_src, te_dst = _pad_edges(tri_edge_index[0], tri_edge_index[1], m, m)
    nt_src, nt_dst = _pad_edges(nt_edge_index[0], nt_edge_index[1], n, m)
    tn_src, tn_dst = _pad_edges(tn_edge_index[0], tn_edge_index[1], m, n)

    p = params['nsa']
    q, k, v = _qkv(xn, p, scale)
    h_node = _sparse_attn_sc(q, k, v, ne_src, ne_dst, False)

    p = params['n2t']
    q = _matmul_bias(xt, p['Wq'] * scale, p['bq'] * scale)
    kv = _matmul_bias(h_node,
                      jnp.concatenate([p['Wk'], p['Wv']], axis=1),
                      jnp.concatenate([p['bk'], p['bv']], axis=0))
    aggr = _sparse_attn_sc(q, kv[:, :D], kv[:, D:], nt_src, nt_dst, False)
    h_tri = _matmul_bias(aggr, p['Wo'], p['bo'], res=xt)

    p = params['tsa']
    q, k, v = _qkv(h_tri, p, d ** 0.5)
    h_tri2 = _sparse_attn_sc(q, k, v, te_src, te_dst, True)

    p = params['t2n']
    q = _matmul_bias(h_node, p['Wq'] * scale, p['bq'] * scale)
    kv = _matmul_bias(h_tri2,
                      jnp.concatenate([p['Wk'], p['Wv']], axis=1),
                      jnp.concatenate([p['bk'], p['bv']], axis=0))
    aggr = _sparse_attn_sc(q, kv[:, :D], kv[:, D:], tn_src, tn_dst, False)
    out_node = _matmul_bias(aggr, p['Wo'], p['bo'], res=h_node)
    return out_node[:n].reshape(t, n, d)


# speedup vs baseline: 1.2455x; 1.0991x over previous
import functools

import jax
import jax.numpy as jnp
import numpy as np
from jax import lax
from jax.experimental import pallas as pl
from jax.experimental.pallas import tpu as pltpu
from jax.experimental.pallas import tpu_sc as plsc

NC, NS, LANES = 2, 16, 16
NW = NC * NS
B = 128
D = 256
NEG = -3.0e38

_MESH = plsc.VectorSubcoreMesh(core_axis_name="c", subcore_axis_name="s")
_SC_PARAMS = pltpu.CompilerParams(needs_layout_passes=False,
                                  use_tc_tiling_on_sc=False)

def _lane_iota():
    return lax.broadcasted_iota(jnp.int32, (LANES,), 0)


def _lane_gather(vec16, idx16):
    dn = lax.GatherDimensionNumbers(offset_dims=(), collapsed_slice_dims=(0,),
                                    start_index_map=(0,))
    return lax.gather(vec16, idx16.reshape(LANES, 1), dn, slice_sizes=(1,),
                      mode=lax.GatherScatterMode.PROMISE_IN_BOUNDS)


def _bcast_lane(vec16, lane, j):
    return _lane_gather(vec16, lane * 0 + j)


def _lane_sum(acc, lane):
    for sh in (1, 2, 4, 8):
        acc = acc + _lane_gather(acc, lane ^ sh)
    return acc


def _mm_body(x_ref, w_ref, b_ref, o_ref):
    o_ref[...] = (
        jnp.dot(x_ref[...], w_ref[...], preferred_element_type=jnp.float32)
        + b_ref[...]
    )


def _mm_res_body(x_ref, w_ref, b_ref, r_ref, o_ref):
    o_ref[...] = (
        jnp.dot(x_ref[...], w_ref[...], preferred_element_type=jnp.float32)
        + b_ref[...] + r_ref[...]
    )


def _matmul_bias(x, w, b, res=None, block_rows=1024):
    r, d = x.shape
    dout = w.shape[1]
    assert r % block_rows == 0, (r, block_rows)
    grid = (r // block_rows,)
    if res is None:
        return pl.pallas_call(
            _mm_body,
            grid=grid,
            in_specs=[
                pl.BlockSpec((block_rows, d), lambda i: (i, 0)),
                pl.BlockSpec((d, dout), lambda i: (0, 0)),
                pl.BlockSpec((1, dout), lambda i: (0, 0)),
            ],
            out_specs=pl.BlockSpec((block_rows, dout), lambda i: (i, 0)),
            out_shape=jax.ShapeDtypeStruct((r, dout), jnp.float32),
        )(x, w, b.reshape(1, dout))
    return pl.pallas_call(
        _mm_res_body,
        grid=grid,
        in_specs=[
            pl.BlockSpec((block_rows, d), lambda i: (i, 0)),
            pl.BlockSpec((d, dout), lambda i: (0, 0)),
            pl.BlockSpec((1, dout), lambda i: (0, 0)),
            pl.BlockSpec((block_rows, dout), lambda i: (i, 0)),
        ],
        out_specs=pl.BlockSpec((block_rows, dout), lambda i: (i, 0)),
        out_shape=jax.ShapeDtypeStruct((r, dout), jnp.float32),
    )(x, w, b.reshape(1, dout), res)


def _qkv(x, p, scale):
    w = jnp.concatenate([p['Wq'] * scale, p['Wk'], p['Wv']], axis=1)
    b = jnp.concatenate([p['bq'] * scale, p['bk'], p['bv']], axis=0)
    out = _matmul_bias(x, w, b)
    return out[:, :D], out[:, D:2 * D], out[:, 2 * D:]


@functools.lru_cache(maxsize=None)
def _k1_logits(EP, NdP, use_max):
    chunk = EP // NW
    nblk = chunk // B

    scratch = [
        pltpu.VMEM((B,), jnp.int32),
        pltpu.VMEM((B,), jnp.int32),
        pltpu.VMEM((B, D), jnp.float32),
        pltpu.VMEM((B, D), jnp.float32),
        pltpu.VMEM((B,), jnp.float32),
    ]
    if use_max:
        scratch.append(pltpu.VMEM((NdP,), jnp.float32))
        out_type = (jax.ShapeDtypeStruct((EP,), jnp.float32),
                    jax.ShapeDtypeStruct((NW, NdP), jnp.float32))
    else:
        out_type = jax.ShapeDtypeStruct((EP,), jnp.float32)

    @functools.partial(pl.kernel, out_type=out_type, mesh=_MESH,
                       scratch_types=scratch, compiler_params=_SC_PARAMS)
    def k1(q_hbm, k_hbm, src_hbm, dst_hbm, *rest):
        if use_max:
            l_hbm, mpart_hbm, src_v, dst_v, q_v, k_v, l_v, m_v = rest
        else:
            l_hbm, src_v, dst_v, q_v, k_v, l_v = rest
        w = lax.axis_index("s") * NC + lax.axis_index("c")
        base = w * chunk

        if use_max:
            def initm(i, carry):
                m_v[pl.ds(i * LANES, LANES)] = jnp.full((LANES,), NEG,
                                                        jnp.float32)
                return carry
            lax.fori_loop(0, NdP // LANES, initm, 0)

        def blk(b, carry):
            off = base + b * B
            pltpu.sync_copy(src_hbm.at[pl.ds(off, B)], src_v)
            pltpu.sync_copy(dst_hbm.at[pl.ds(off, B)], dst_v)
            pltpu.sync_copy(q_hbm.at[dst_v], q_v)
            pltpu.sync_copy(k_hbm.at[src_v], k_v)

            def dotgrp(g, ecarry):
                lane = _lane_iota()
                lvec = jnp.zeros((LANES,), jnp.float32)
                for j in range(LANES):
                    e = g * LANES + j
                    acc = q_v[e, pl.ds(0, LANES)] * k_v[e, pl.ds(0, LANES)]
                    for c in range(1, D // LANES):
                        acc = acc + (q_v[e, pl.ds(c * LANES, LANES)]
                                     * k_v[e, pl.ds(c * LANES, LANES)])
                    lvec = jnp.where(lane == j, _lane_sum(acc, lane), lvec)
                l_v[pl.ds(g * LANES, LANES)] = lvec
                return ecarry
            lax.fori_loop(0, B // LANES, dotgrp, 0)

            if use_max:
                def grp(g, gcarry):
                    sl = pl.ds(g * LANES, LANES)
                    l16 = l_v[sl]
                    d16 = dst_v[sl]

                    def cond(c_):
                        return c_

                    def body(c_):
                        mo = plsc.load_gather(m_v, [d16])
                        plsc.store_scatter(m_v, [d16], l16, mask=l16 > mo)
                        mo2 = plsc.load_gather(m_v, [d16])
                        return jnp.any(l16 > mo2)
                    lax.while_loop(cond, body, True)
                    return gcarry
                lax.fori_loop(0, B // LANES, grp, 0)

            pltpu.sync_copy(l_v, l_hbm.at[pl.ds(off, B)])
            return carry
        lax.fori_loop(0, nblk, blk, 0)

        if use_max:
            pltpu.sync_copy(m_v, mpart_hbm.at[w])
    return k1


BE = 64


@functools.lru_cache(maxsize=None)
def _k1_fused(EP, NdP):
    chunk = EP // NW
    nblk = chunk // BE
    sl16 = NdP // NS
    assert nblk % 2 == 0

    scratch = [
        pltpu.VMEM((2 * BE,), jnp.int32),
        pltpu.VMEM((2 * BE,), jnp.int32),
        pltpu.VMEM((2 * BE, D), jnp.float32),
        pltpu.VMEM((2 * BE, D), jnp.float32),
        pltpu.VMEM((BE,), jnp.int32),
        pltpu.VMEM((BE,), jnp.float32),
        pltpu.VMEM((sl16,), jnp.float32),
        pltpu.VMEM_SHARED((NdP,), jnp.float32),
        pltpu.SemaphoreType.DMA,
    ]
    out_type = (jax.ShapeDtypeStruct((EP,), jnp.float32),
                jax.ShapeDtypeStruct((NC, NdP), jnp.float32))

    @functools.partial(pl.kernel, out_type=out_type, mesh=_MESH,
                       scratch_types=scratch, compiler_params=_SC_PARAMS)
    def k1e(qk_hbm, sd_hbm, dst_hbm, e_hbm, spart_hbm,
            i0, i1, qk0, qk1, dst_v, e_v, z_v, s_sh, gsem):
        c = lax.axis_index("c")
        s = lax.axis_index("s")
        w = s * NC + c
        base = w * chunk
        gblk = base // BE
        ibufs = (i0, i1)
        qkbufs = (qk0, qk1)

        def zb(i, carry):
            z_v[pl.ds(i * LANES, LANES)] = jnp.zeros((LANES,), jnp.float32)
            return carry
        lax.fori_loop(0, sl16 // LANES, zb, 0)
        pltpu.sync_copy(z_v, s_sh.at[pl.ds(s * sl16, sl16)])
        plsc.subcore_barrier()

        pltpu.sync_copy(sd_hbm.at[gblk], i0)
        pltpu.async_copy(qk_hbm.at[i0], qk0, gsem)

        def pair(g, carry):
            for b01 in range(2):
                t = g * 2 + b01
                icur, qkc = ibufs[b01], qkbufs[b01]
                inxt, qkn = ibufs[1 - b01], qkbufs[1 - b01]
                pltpu.make_async_copy(qk_hbm.at[pl.ds(0, 2 * BE)], qkc,
                                      gsem).wait()

                @pl.when(t + 1 < nblk)
                def _():
                    pltpu.sync_copy(sd_hbm.at[gblk + t + 1], inxt)
                    pltpu.async_copy(qk_hbm.at[inxt], qkn, gsem)

                off = base + t * BE
                pltpu.sync_copy(dst_hbm.at[pl.ds(off, BE)], dst_v)

                def dotgrp(gg, ecarry):
                    lane = _lane_iota()
                    lvec = jnp.zeros((LANES,), jnp.float32)
                    for j in range(LANES):
                        e = gg * LANES + j
                        acc = (qkc[e, pl.ds(0, LANES)]
                               * qkc[BE + e, pl.ds(0, LANES)])
                        for cc in range(1, D // LANES):
                            acc = acc + (qkc[e, pl.ds(cc * LANES, LANES)]
                                         * qkc[BE + e,
                                               pl.ds(cc * LANES, LANES)])
                        lvec = jnp.where(lane == j, _lane_sum(acc, lane),
                                         lvec)
                    e_v[pl.ds(gg * LANES, LANES)] = jnp.exp(lvec)
                    return ecarry
                lax.fori_loop(0, BE // LANES, dotgrp, 0)

                pltpu.sync_copy(e_v, e_hbm.at[pl.ds(off, BE)])
                pltpu.sync_copy(e_v, s_sh.at[dst_v], add=True)
            return carry
        lax.fori_loop(0, nblk // 2, pair, 0)

        plsc.subcore_barrier()
        pltpu.sync_copy(s_sh.at[pl.ds(s * sl16, sl16)],
                        spart_hbm.at[c, pl.ds(s * sl16, sl16)])
    return k1e


@functools.lru_cache(maxsize=None)
def _k2_merge(NdP):
    sl_len = NdP // NW

    @functools.partial(
        pl.kernel,
        out_type=jax.ShapeDtypeStruct((NdP,), jnp.float32),
        mesh=_MESH,
        scratch_types=[pltpu.VMEM((sl_len,), jnp.float32),
                       pltpu.VMEM((sl_len,), jnp.float32)],
        compiler_params=_SC_PARAMS)
    def k2(mpart_hbm, mfin_hbm, acc_v, tmp_v):
        w = lax.axis_index("s") * NC + lax.axis_index("c")
        off = w * sl_len
        pltpu.sync_copy(mpart_hbm.at[0, pl.ds(off, sl_len)], acc_v)

        def red(w2, carry):
            pltpu.sync_copy(mpart_hbm.at[w2, pl.ds(off, sl_len)], tmp_v)

            def ch(i, icarry):
                s_ = pl.ds(i * LANES, LANES)
                acc_v[s_] = jnp.maximum(acc_v[s_], tmp_v[s_])
                return icarry
            lax.fori_loop(0, sl_len // LANES, ch, 0)
            return carry
        lax.fori_loop(1, NW, red, 0)
        pltpu.sync_copy(acc_v, mfin_hbm.at[pl.ds(off, sl_len)])
    return k2


@functools.lru_cache(maxsize=None)
def _k3_expsum(EP, NdP, use_max):
    chunk = EP // NW
    nblk = chunk // B
    sl16 = NdP // NS

    scratch = [
        pltpu.VMEM((B,), jnp.int32),
        pltpu.VMEM((B,), jnp.float32),
        pltpu.VMEM((B,), jnp.float32),
        pltpu.VMEM((sl16,), jnp.float32),
        pltpu.VMEM_SHARED((NdP,), jnp.float32),
    ]
    if use_max:
        scratch.append(pltpu.VMEM((NdP,), jnp.float32))

    out_type = (jax.ShapeDtypeStruct((EP,), jnp.float32),
                jax.ShapeDtypeStruct((NC, NdP), jnp.float32))

    @functools.partial(pl.kernel, out_type=out_type, mesh=_MESH,
                       scratch_types=scratch, compiler_params=_SC_PARAMS)
    def k3(dst_hbm, l_hbm, *rest):
        if use_max:
            mfin_hbm, e_hbm, spart_hbm, dst_v, l_v, e_v, z_v, s_sh, m_v = rest
        else:
            e_hbm, spart_hbm, dst_v, l_v, e_v, z_v, s_sh = rest
        c = lax.axis_index("c")
        s = lax.axis_index("s")
        w = s * NC + c

        def zb(i, carry):
            z_v[pl.ds(i * LANES, LANES)] = jnp.zeros((LANES,), jnp.float32)
            return carry
        lax.fori_loop(0, sl16 // LANES, zb, 0)
        pltpu.sync_copy(z_v, s_sh.at[pl.ds(s * sl16, sl16)])
        if use_max:
            pltpu.sync_copy(mfin_hbm, m_v)
        plsc.subcore_barrier()

        def blk(b, carry):
            off = w * chunk + b * B
            pltpu.sync_copy(dst_hbm.at[pl.ds(off, B)], dst_v)
            pltpu.sync_copy(l_hbm.at[pl.ds(off, B)], l_v)

            def grp(g, gcarry):
                sl = pl.ds(g * LANES, LANES)
                l16 = l_v[sl]
                if use_max:
                    m16 = plsc.load_gather(m_v, [dst_v[sl]])
                    e_v[sl] = jnp.exp(l16 - m16)
                else:
                    e_v[sl] = jnp.exp(l16)
                return gcarry
            lax.fori_loop(0, B // LANES, grp, 0)
            pltpu.sync_copy(e_v, e_hbm.at[pl.ds(off, B)])
            pltpu.sync_copy(e_v, s_sh.at[dst_v], add=True)
            return carry
        lax.fori_loop(0, nblk, blk, 0)

        plsc.subcore_barrier()
        pltpu.sync_copy(s_sh.at[pl.ds(s * sl16, sl16)],
                        spart_hbm.at[c, pl.ds(s * sl16, sl16)])
    return k3


B5 = 64


@functools.lru_cache(maxsize=None)
def _k5_scatter(EP, NdP, DSUB):
    npass = D // DSUB
    chunk = EP // NW
    nblk = chunk // B5
    rows16 = NdP // NS
    assert nblk % 2 == 0 and rows16 % B5 == 0

    scratch = [
        pltpu.VMEM((B5,), jnp.int32),
        pltpu.VMEM((B5,), jnp.int32),
        pltpu.VMEM((B5,), jnp.int32),
        pltpu.VMEM((B5,), jnp.int32),
        pltpu.VMEM((B5,), jnp.float32),
        pltpu.VMEM((B5, DSUB), jnp.float32),
        pltpu.VMEM((B5, DSUB), jnp.float32),
        pltpu.VMEM((B5, DSUB), jnp.float32),
        pltpu.VMEM((B5, DSUB), jnp.float32),
        pltpu.VMEM((NdP,), jnp.float32),
        pltpu.VMEM_SHARED((NdP, DSUB), jnp.float32),
        pltpu.SemaphoreType.DMA,
        pltpu.SemaphoreType.DMA,
    ]

    @functools.partial(
        pl.kernel,
        out_type=jax.ShapeDtypeStruct((npass * NC, NdP, DSUB), jnp.float32),
        mesh=_MESH, scratch_types=scratch, compiler_params=_SC_PARAMS)
    def k5(*args):
        vq = args[:npass]
        (src_hbm, dst_hbm, e_hbm, s_hbm, out_hbm,
         sr0, sr1, dr0, dr1, e_v, v0, v1, c0, c1, s_v, slab,
         gsem, ssem) = args[npass:]
        c = lax.axis_index("c")
        s = lax.axis_index("s")
        w = s * NC + c
        base = w * chunk
        srcb = (sr0, sr1)
        dstb = (dr0, dr1)
        vrb = (v0, v1)
        scb = (c0, c1)

        pltpu.sync_copy(s_hbm, s_v)

        def zrow(i, carry):
            for c16 in range(DSUB // LANES):
                c0[i, pl.ds(c16 * LANES, LANES)] = jnp.zeros(
                    (LANES,), jnp.float32)
            return carry

        for p in range(npass):
            lax.fori_loop(0, B5, zrow, 0)
            for q0 in range(0, rows16, B5):
                pltpu.sync_copy(c0,
                                slab.at[pl.ds(s * rows16 + q0, B5)])
            plsc.subcore_barrier()

            pltpu.sync_copy(src_hbm.at[pl.ds(base, B5)], sr0)
            pltpu.async_copy(vq[p].at[sr0], v0, gsem)

            def pair(g, carry):
                for b01 in range(2):
                    t = g * 2 + b01
                    off = base + t * B5
                    pltpu.make_async_copy(vq[p].at[pl.ds(0, B5)],
                                          vrb[b01], gsem).wait()

                    @pl.when(t + 1 < nblk)
                    def _():
                        pltpu.sync_copy(src_hbm.at[pl.ds(off + B5, B5)],
                                        srcb[1 - b01])
                        pltpu.async_copy(vq[p].at[srcb[1 - b01]],
                                         vrb[1 - b01], gsem)

                    @pl.when(t >= 2)
                    def _():
                        pltpu.make_async_copy(vq[p].at[pl.ds(0, B5)],
                                              scb[b01], ssem).wait()

                    pltpu.sync_copy(dst_hbm.at[pl.ds(off, B5)], dstb[b01])
                    pltpu.sync_copy(e_hbm.at[pl.ds(off, B5)], e_v)

                    def grp(g2, gcarry):
                        lane = _lane_iota()
                        sl = pl.ds(g2 * LANES, LANES)
                        d16 = dstb[b01][sl]
                        sv = plsc.load_gather(s_v, [d16])
                        a16 = e_v[sl] / (sv + 1e-16)
                        for j in range(LANES):
                            e = g2 * LANES + j
                            va = _bcast_lane(a16, lane, j)
                            for c16 in range(DSUB // LANES):
                                slc = pl.ds(c16 * LANES, LANES)
                                scb[b01][e, slc] = vrb[b01][e, slc] * va
                        return gcarry
                    lax.fori_loop(0, B5 // LANES, grp, 0)

                    pltpu.async_copy(scb[b01], slab.at[dstb[b01]], ssem,
                                     add=True)
                return carry
            lax.fori_loop(0, nblk // 2, pair, 0)

            pltpu.make_async_copy(vq[p].at[pl.ds(0, B5)], c0, ssem).wait()
            pltpu.make_async_copy(vq[p].at[pl.ds(0, B5)], c1, ssem).wait()

            plsc.subcore_barrier()
            out_row = p * NC + c
            for q0 in range(0, rows16, B5):
                pltpu.sync_copy(slab.at[pl.ds(s * rows16 + q0, B5)],
                                out_hbm.at[out_row,
                                           pl.ds(s * rows16 + q0, B5)])
            plsc.subcore_barrier()
    return k5


def _add_body(a_ref, b_ref, o_ref):
    o_ref[...] = a_ref[...] + b_ref[...]


def _pallas_add(a, b, block_rows=1024):
    r, d = a.shape
    grid = (r // block_rows,)
    return pl.pallas_call(
        _add_body,
        grid=grid,
        in_specs=[pl.BlockSpec((block_rows, d), lambda i: (i, 0)),
                  pl.BlockSpec((block_rows, d), lambda i: (i, 0))],
        out_specs=pl.BlockSpec((block_rows, d), lambda i: (i, 0)),
        out_shape=jax.ShapeDtypeStruct((r, d), jnp.float32),
    )(a, b)


def _round_up(x, m):
    return (x + m - 1) // m * m


def _sparse_attn_sc(q, k, v, src, dst, use_max):
    NdP = q.shape[0]
    EP = src.shape[0]
    if use_max:
        l, mpart = _k1_logits(EP, NdP, True)(q, k, src, dst)
        mfin = _k2_merge(NdP)(mpart)
        e, spart = _k3_expsum(EP, NdP, True)(dst, l, mfin)
    else:
        qk = jnp.concatenate([q, k], axis=0)
        sd = jnp.concatenate([dst.reshape(-1, BE),
                              src.reshape(-1, BE) + NdP], axis=1)
        e, spart = _k1_fused(EP, NdP)(qk, sd, dst)
    dsub = 128 if NdP <= 12288 else 64
    npass = D // dsub
    vq = [v[:, i * dsub:(i + 1) * dsub] for i in range(npass)]
    srows = NdP // 128
    s = _pallas_add(spart[0].reshape(srows, 128),
                    spart[1].reshape(srows, 128),
                    block_rows=srows).reshape(NdP)
    outp = _k5_scatter(EP, NdP, dsub)(*vq, src, dst, e, s)
    a = jnp.transpose(outp[0::NC], (1, 0, 2)).reshape(NdP, D)
    b = jnp.transpose(outp[1::NC], (1, 0, 2)).reshape(NdP, D)
    return _pallas_add(a, b)


def _pad_edges(src, dst, ns, nd):
    E = src.shape[0]
    EP = _round_up(E, NW * B)
    src = jnp.pad(src.astype(jnp.int32), (0, EP - E), constant_values=ns)
    dst = jnp.pad(dst.astype(jnp.int32), (0, EP - E), constant_values=nd)
    return src, dst


def kernel(x_node, x_tri, params, node_edge_index, tri_edge_index,
           nt_edge_index, tn_edge_index):
    t, n, d = x_node.shape
    _, m, _ = x_tri.shape
    assert t == 1 and d == D
    NP = _round_up(n, 1024)
    MP = _round_up(m, 1024)
    scale = d ** (-0.5)

    xn = jnp.pad(x_node.reshape(n, d), ((0, NP - n), (0, 0)))
    xt = jnp.pad(x_tri.reshape(m, d), ((0, MP - m), (0, 0)))

    ne_src, ne_dst = _pad_edges(node_edge_index[0], node_edge_index[1], n, n)
    te_src, te_dst = _pad_edges(tri_edge_index[0], tri_edge_index[1], m, m)
    nt_src, nt_dst = _pad_edges(nt_edge_index[0], nt_edge_index[1], n, m)
    tn_src, tn_dst = _pad_edges(tn_edge_index[0], tn_edge_index[1], m, n)

    p = params['nsa']
    q, k, v = _qkv(xn, p, scale)
    h_node = _sparse_attn_sc(q, k, v, ne_src, ne_dst, False)

    p = params['n2t']
    q = _matmul_bias(xt, p['Wq'] * scale, p['bq'] * scale)
    kv = _matmul_bias(h_node,
                      jnp.concatenate([p['Wk'], p['Wv']], axis=1),
                      jnp.concatenate([p['bk'], p['bv']], axis=0))
    aggr = _sparse_attn_sc(q, kv[:, :D], kv[:, D:], nt_src, nt_dst, False)
    h_tri = _matmul_bias(aggr, p['Wo'], p['bo'], res=xt)

    p = params['tsa']
    q, k, v = _qkv(h_tri, p, d ** 0.5)
    h_tri2 = _sparse_attn_sc(q, k, v, te_src, te_dst, True)

    p = params['t2n']
    q = _matmul_bias(h_node, p['Wq'] * scale, p['bq'] * scale)
    kv = _matmul_bias(h_tri2,
                      jnp.concatenate([p['Wk'], p['Wv']], axis=1),
                      jnp.concatenate([p['bk'], p['bv']], axis=0))
    aggr = _sparse_attn_sc(q, kv[:, :D], kv[:, D:], tn_src, tn_dst, False)
    out_node = _matmul_bias(aggr, p['Wo'], p['bo'], res=h_node)
    return out_node[:n].reshape(t, n, d)


# speedup vs baseline: 1.3542x; 1.0873x over previous
import functools

import jax
import jax.numpy as jnp
import numpy as np
from jax import lax
from jax.experimental import pallas as pl
from jax.experimental.pallas import tpu as pltpu
from jax.experimental.pallas import tpu_sc as plsc

NC, NS, LANES = 2, 16, 16
NW = NC * NS
B = 128
D = 256
NEG = -3.0e38

_MESH = plsc.VectorSubcoreMesh(core_axis_name="c", subcore_axis_name="s")
_SC_PARAMS = pltpu.CompilerParams(needs_layout_passes=False,
                                  use_tc_tiling_on_sc=False)

def _lane_iota():
    return lax.broadcasted_iota(jnp.int32, (LANES,), 0)


def _lane_gather(vec16, idx16):
    dn = lax.GatherDimensionNumbers(offset_dims=(), collapsed_slice_dims=(0,),
                                    start_index_map=(0,))
    return lax.gather(vec16, idx16.reshape(LANES, 1), dn, slice_sizes=(1,),
                      mode=lax.GatherScatterMode.PROMISE_IN_BOUNDS)


def _bcast_lane(vec16, lane, j):
    return _lane_gather(vec16, lane * 0 + j)


def _lane_sum(acc, lane):
    for sh in (1, 2, 4, 8):
        acc = acc + _lane_gather(acc, lane ^ sh)
    return acc


def _mm_body(x_ref, w_ref, b_ref, o_ref):
    o_ref[...] = (
        jnp.dot(x_ref[...], w_ref[...], preferred_element_type=jnp.float32)
        + b_ref[...]
    )


def _mm_res_body(x_ref, w_ref, b_ref, r_ref, o_ref):
    o_ref[...] = (
        jnp.dot(x_ref[...], w_ref[...], preferred_element_type=jnp.float32)
        + b_ref[...] + r_ref[...]
    )


def _matmul_bias(x, w, b, res=None, block_rows=1024):
    r, d = x.shape
    dout = w.shape[1]
    assert r % block_rows == 0, (r, block_rows)
    grid = (r // block_rows,)
    if res is None:
        return pl.pallas_call(
            _mm_body,
            grid=grid,
            in_specs=[
                pl.BlockSpec((block_rows, d), lambda i: (i, 0)),
                pl.BlockSpec((d, dout), lambda i: (0, 0)),
                pl.BlockSpec((1, dout), lambda i: (0, 0)),
            ],
            out_specs=pl.BlockSpec((block_rows, dout), lambda i: (i, 0)),
            out_shape=jax.ShapeDtypeStruct((r, dout), jnp.float32),
        )(x, w, b.reshape(1, dout))
    return pl.pallas_call(
        _mm_res_body,
        grid=grid,
        in_specs=[
            pl.BlockSpec((block_rows, d), lambda i: (i, 0)),
            pl.BlockSpec((d, dout), lambda i: (0, 0)),
            pl.BlockSpec((1, dout), lambda i: (0, 0)),
            pl.BlockSpec((block_rows, dout), lambda i: (i, 0)),
        ],
        out_specs=pl.BlockSpec((block_rows, dout), lambda i: (i, 0)),
        out_shape=jax.ShapeDtypeStruct((r, dout), jnp.float32),
    )(x, w, b.reshape(1, dout), res)


def _qkv(x, p, scale):
    w = jnp.concatenate([p['Wq'] * scale, p['Wk'], p['Wv']], axis=1)
    b = jnp.concatenate([p['bq'] * scale, p['bk'], p['bv']], axis=0)
    out = _matmul_bias(x, w, b)
    return out[:, :D], out[:, D:2 * D], out[:, 2 * D:]


@functools.lru_cache(maxsize=None)
def _k1_max(EP, NdP):
    BE = 64
    chunk = EP // NW
    nblk = chunk // BE
    assert nblk % 2 == 0

    scratch = [
        pltpu.VMEM((2 * BE,), jnp.int32),
        pltpu.VMEM((2 * BE,), jnp.int32),
        pltpu.VMEM((2 * BE, D), jnp.float32),
        pltpu.VMEM((2 * BE, D), jnp.float32),
        pltpu.VMEM((BE,), jnp.int32),
        pltpu.VMEM((BE,), jnp.float32),
        pltpu.VMEM((NdP,), jnp.float32),
        pltpu.SemaphoreType.DMA,
    ]
    out_type = (jax.ShapeDtypeStruct((EP,), jnp.float32),
                jax.ShapeDtypeStruct((NW, NdP), jnp.float32))

    @functools.partial(pl.kernel, out_type=out_type, mesh=_MESH,
                       scratch_types=scratch, compiler_params=_SC_PARAMS)
    def k1m(qk_hbm, sd_hbm, dst_hbm, l_hbm, mpart_hbm,
            i0, i1, qk0, qk1, dst_v, l_v, m_v, gsem):
        c = lax.axis_index("c")
        s = lax.axis_index("s")
        w = s * NC + c
        base = w * chunk
        gblk = base // BE
        ibufs = (i0, i1)
        qkbufs = (qk0, qk1)

        def initm(i, carry):
            m_v[pl.ds(i * LANES, LANES)] = jnp.full((LANES,), NEG,
                                                    jnp.float32)
            return carry
        lax.fori_loop(0, NdP // LANES, initm, 0)

        pltpu.sync_copy(sd_hbm.at[gblk], i0)
        pltpu.async_copy(qk_hbm.at[i0], qk0, gsem)

        def pair(g, carry):
            for b01 in range(2):
                t = g * 2 + b01
                icur, qkc = ibufs[b01], qkbufs[b01]
                inxt, qkn = ibufs[1 - b01], qkbufs[1 - b01]
                pltpu.make_async_copy(qk_hbm.at[pl.ds(0, 2 * BE)], qkc,
                                      gsem).wait()

                @pl.when(t + 1 < nblk)
                def _():
                    pltpu.sync_copy(sd_hbm.at[gblk + t + 1], inxt)
                    pltpu.async_copy(qk_hbm.at[inxt], qkn, gsem)

                off = base + t * BE
                pltpu.sync_copy(dst_hbm.at[pl.ds(off, BE)], dst_v)

                def dotgrp(gg, ecarry):
                    lane = _lane_iota()
                    lvec = jnp.zeros((LANES,), jnp.float32)
                    for j in range(LANES):
                        e = gg * LANES + j
                        acc = (qkc[e, pl.ds(0, LANES)]
                               * qkc[BE + e, pl.ds(0, LANES)])
                        for cc in range(1, D // LANES):
                            acc = acc + (qkc[e, pl.ds(cc * LANES, LANES)]
                                         * qkc[BE + e,
                                               pl.ds(cc * LANES, LANES)])
                        lvec = jnp.where(lane == j, _lane_sum(acc, lane),
                                         lvec)
                    l_v[pl.ds(gg * LANES, LANES)] = lvec
                    return ecarry
                lax.fori_loop(0, BE // LANES, dotgrp, 0)

                def grp(gg, gcarry):
                    sl = pl.ds(gg * LANES, LANES)
                    l16 = l_v[sl]
                    d16 = dst_v[sl]

                    def cond(c_):
                        return c_

                    def body(c_):
                        mo = plsc.load_gather(m_v, [d16])
                        plsc.store_scatter(m_v, [d16], l16, mask=l16 > mo)
                        mo2 = plsc.load_gather(m_v, [d16])
                        return jnp.any(l16 > mo2)
                    lax.while_loop(cond, body, True)
                    return gcarry
                lax.fori_loop(0, BE // LANES, grp, 0)

                pltpu.sync_copy(l_v, l_hbm.at[pl.ds(off, BE)])
            return carry
        lax.fori_loop(0, nblk // 2, pair, 0)

        pltpu.sync_copy(m_v, mpart_hbm.at[w])
    return k1m


BE = 64


@functools.lru_cache(maxsize=None)
def _k1_fused(EP, NdP):
    chunk = EP // NW
    nblk = chunk // BE
    sl16 = NdP // NS
    assert nblk % 2 == 0

    scratch = [
        pltpu.VMEM((2 * BE,), jnp.int32),
        pltpu.VMEM((2 * BE,), jnp.int32),
        pltpu.VMEM((2 * BE, D), jnp.float32),
        pltpu.VMEM((2 * BE, D), jnp.float32),
        pltpu.VMEM((BE,), jnp.int32),
        pltpu.VMEM((BE,), jnp.float32),
        pltpu.VMEM((sl16,), jnp.float32),
        pltpu.VMEM_SHARED((NdP,), jnp.float32),
        pltpu.SemaphoreType.DMA,
    ]
    out_type = (jax.ShapeDtypeStruct((EP,), jnp.float32),
                jax.ShapeDtypeStruct((NC, NdP), jnp.float32))

    @functools.partial(pl.kernel, out_type=out_type, mesh=_MESH,
                       scratch_types=scratch, compiler_params=_SC_PARAMS)
    def k1e(qk_hbm, sd_hbm, dst_hbm, e_hbm, spart_hbm,
            i0, i1, qk0, qk1, dst_v, e_v, z_v, s_sh, gsem):
        c = lax.axis_index("c")
        s = lax.axis_index("s")
        w = s * NC + c
        base = w * chunk
        gblk = base // BE
        ibufs = (i0, i1)
        qkbufs = (qk0, qk1)

        def zb(i, carry):
            z_v[pl.ds(i * LANES, LANES)] = jnp.zeros((LANES,), jnp.float32)
            return carry
        lax.fori_loop(0, sl16 // LANES, zb, 0)
        pltpu.sync_copy(z_v, s_sh.at[pl.ds(s * sl16, sl16)])
        plsc.subcore_barrier()

        pltpu.sync_copy(sd_hbm.at[gblk], i0)
        pltpu.async_copy(qk_hbm.at[i0], qk0, gsem)

        def pair(g, carry):
            for b01 in range(2):
                t = g * 2 + b01
                icur, qkc = ibufs[b01], qkbufs[b01]
                inxt, qkn = ibufs[1 - b01], qkbufs[1 - b01]
                pltpu.make_async_copy(qk_hbm.at[pl.ds(0, 2 * BE)], qkc,
                                      gsem).wait()

                @pl.when(t + 1 < nblk)
                def _():
                    pltpu.sync_copy(sd_hbm.at[gblk + t + 1], inxt)
                    pltpu.async_copy(qk_hbm.at[inxt], qkn, gsem)

                off = base + t * BE
                pltpu.sync_copy(dst_hbm.at[pl.ds(off, BE)], dst_v)

                def dotgrp(gg, ecarry):
                    lane = _lane_iota()
                    lvec = jnp.zeros((LANES,), jnp.float32)
                    for j in range(LANES):
                        e = gg * LANES + j
                        acc = (qkc[e, pl.ds(0, LANES)]
                               * qkc[BE + e, pl.ds(0, LANES)])
                        for cc in range(1, D // LANES):
                            acc = acc + (qkc[e, pl.ds(cc * LANES, LANES)]
                                         * qkc[BE + e,
                                               pl.ds(cc * LANES, LANES)])
                        lvec = jnp.where(lane == j, _lane_sum(acc, lane),
                                         lvec)
                    e_v[pl.ds(gg * LANES, LANES)] = jnp.exp(lvec)
                    return ecarry
                lax.fori_loop(0, BE // LANES, dotgrp, 0)

                pltpu.sync_copy(e_v, e_hbm.at[pl.ds(off, BE)])
                pltpu.sync_copy(e_v, s_sh.at[dst_v], add=True)
            return carry
        lax.fori_loop(0, nblk // 2, pair, 0)

        plsc.subcore_barrier()
        pltpu.sync_copy(s_sh.at[pl.ds(s * sl16, sl16)],
                        spart_hbm.at[c, pl.ds(s * sl16, sl16)])
    return k1e


@functools.lru_cache(maxsize=None)
def _k2_merge(NdP):
    sl_len = NdP // NW

    @functools.partial(
        pl.kernel,
        out_type=jax.ShapeDtypeStruct((NdP,), jnp.float32),
        mesh=_MESH,
        scratch_types=[pltpu.VMEM((sl_len,), jnp.float32),
                       pltpu.VMEM((sl_len,), jnp.float32)],
        compiler_params=_SC_PARAMS)
    def k2(mpart_hbm, mfin_hbm, acc_v, tmp_v):
        w = lax.axis_index("s") * NC + lax.axis_index("c")
        off = w * sl_len
        pltpu.sync_copy(mpart_hbm.at[0, pl.ds(off, sl_len)], acc_v)

        def red(w2, carry):
            pltpu.sync_copy(mpart_hbm.at[w2, pl.ds(off, sl_len)], tmp_v)

            def ch(i, icarry):
                s_ = pl.ds(i * LANES, LANES)
                acc_v[s_] = jnp.maximum(acc_v[s_], tmp_v[s_])
                return icarry
            lax.fori_loop(0, sl_len // LANES, ch, 0)
            return carry
        lax.fori_loop(1, NW, red, 0)
        pltpu.sync_copy(acc_v, mfin_hbm.at[pl.ds(off, sl_len)])
    return k2


@functools.lru_cache(maxsize=None)
def _k3_expsum(EP, NdP, use_max):
    chunk = EP // NW
    nblk = chunk // B
    sl16 = NdP // NS

    scratch = [
        pltpu.VMEM((B,), jnp.int32),
        pltpu.VMEM((B,), jnp.float32),
        pltpu.VMEM((B,), jnp.float32),
        pltpu.VMEM((sl16,), jnp.float32),
        pltpu.VMEM_SHARED((NdP,), jnp.float32),
    ]
    if use_max:
        scratch.append(pltpu.VMEM((NdP,), jnp.float32))

    out_type = (jax.ShapeDtypeStruct((EP,), jnp.float32),
                jax.ShapeDtypeStruct((NC, NdP), jnp.float32))

    @functools.partial(pl.kernel, out_type=out_type, mesh=_MESH,
                       scratch_types=scratch, compiler_params=_SC_PARAMS)
    def k3(dst_hbm, l_hbm, *rest):
        if use_max:
            mfin_hbm, e_hbm, spart_hbm, dst_v, l_v, e_v, z_v, s_sh, m_v = rest
        else:
            e_hbm, spart_hbm, dst_v, l_v, e_v, z_v, s_sh = rest
        c = lax.axis_index("c")
        s = lax.axis_index("s")
        w = s * NC + c

        def zb(i, carry):
            z_v[pl.ds(i * LANES, LANES)] = jnp.zeros((LANES,), jnp.float32)
            return carry
        lax.fori_loop(0, sl16 // LANES, zb, 0)
        pltpu.sync_copy(z_v, s_sh.at[pl.ds(s * sl16, sl16)])
        if use_max:
            pltpu.sync_copy(mfin_hbm, m_v)
        plsc.subcore_barrier()

        def blk(b, carry):
            off = w * chunk + b * B
            pltpu.sync_copy(dst_hbm.at[pl.ds(off, B)], dst_v)
            pltpu.sync_copy(l_hbm.at[pl.ds(off, B)], l_v)

            def grp(g, gcarry):
                sl = pl.ds(g * LANES, LANES)
                l16 = l_v[sl]
                if use_max:
                    m16 = plsc.load_gather(m_v, [dst_v[sl]])
                    e_v[sl] = jnp.exp(l16 - m16)
                else:
                    e_v[sl] = jnp.exp(l16)
                return gcarry
            lax.fori_loop(0, B // LANES, grp, 0)
            pltpu.sync_copy(e_v, e_hbm.at[pl.ds(off, B)])
            pltpu.sync_copy(e_v, s_sh.at[dst_v], add=True)
            return carry
        lax.fori_loop(0, nblk, blk, 0)

        plsc.subcore_barrier()
        pltpu.sync_copy(s_sh.at[pl.ds(s * sl16, sl16)],
                        spart_hbm.at[c, pl.ds(s * sl16, sl16)])
    return k3


B5 = 64


@functools.lru_cache(maxsize=None)
def _k5_scatter(EP, NdP, DSUB):
    npass = D // DSUB
    chunk = EP // NW
    nblk = chunk // B5
    rows16 = NdP // NS
    assert nblk % 2 == 0 and rows16 % B5 == 0

    scratch = [
        pltpu.VMEM((B5,), jnp.int32),
        pltpu.VMEM((B5,), jnp.int32),
        pltpu.VMEM((B5,), jnp.int32),
        pltpu.VMEM((B5,), jnp.int32),
        pltpu.VMEM((B5,), jnp.float32),
        pltpu.VMEM((B5, DSUB), jnp.float32),
        pltpu.VMEM((B5, DSUB), jnp.float32),
        pltpu.VMEM((B5, DSUB), jnp.float32),
        pltpu.VMEM((B5, DSUB), jnp.float32),
        pltpu.VMEM((NdP,), jnp.float32),
        pltpu.VMEM_SHARED((NdP, DSUB), jnp.float32),
        pltpu.SemaphoreType.DMA,
        pltpu.SemaphoreType.DMA,
    ]

    @functools.partial(
        pl.kernel,
        out_type=jax.ShapeDtypeStruct((npass * NC, NdP, DSUB), jnp.float32),
        mesh=_MESH, scratch_types=scratch, compiler_params=_SC_PARAMS)
    def k5(*args):
        vq = args[:npass]
        (src_hbm, dst_hbm, e_hbm, s_hbm, out_hbm,
         sr0, sr1, dr0, dr1, e_v, v0, v1, c0, c1, s_v, slab,
         gsem, ssem) = args[npass:]
        c = lax.axis_index("c")
        s = lax.axis_index("s")
        w = s * NC + c
        base = w * chunk
        srcb = (sr0, sr1)
        dstb = (dr0, dr1)
        vrb = (v0, v1)
        scb = (c0, c1)

        pltpu.sync_copy(s_hbm, s_v)

        def zrow(i, carry):
            for c16 in range(DSUB // LANES):
                c0[i, pl.ds(c16 * LANES, LANES)] = jnp.zeros(
                    (LANES,), jnp.float32)
            return carry

        for p in range(npass):
            lax.fori_loop(0, B5, zrow, 0)
            for q0 in range(0, rows16, B5):
                pltpu.sync_copy(c0,
                                slab.at[pl.ds(s * rows16 + q0, B5)])
            plsc.subcore_barrier()

            pltpu.sync_copy(src_hbm.at[pl.ds(base, B5)], sr0)
            pltpu.async_copy(vq[p].at[sr0], v0, gsem)

            def pair(g, carry):
                for b01 in range(2):
                    t = g * 2 + b01
                    off = base + t * B5
                    pltpu.make_async_copy(vq[p].at[pl.ds(0, B5)],
                                          vrb[b01], gsem).wait()

                    @pl.when(t + 1 < nblk)
                    def _():
                        pltpu.sync_copy(src_hbm.at[pl.ds(off + B5, B5)],
                                        srcb[1 - b01])
                        pltpu.async_copy(vq[p].at[srcb[1 - b01]],
                                         vrb[1 - b01], gsem)

                    @pl.when(t >= 2)
                    def _():
                        pltpu.make_async_copy(vq[p].at[pl.ds(0, B5)],
                                              scb[b01], ssem).wait()

                    pltpu.sync_copy(dst_hbm.at[pl.ds(off, B5)], dstb[b01])
                    pltpu.sync_copy(e_hbm.at[pl.ds(off, B5)], e_v)

                    def grp(g2, gcarry):
                        lane = _lane_iota()
                        sl = pl.ds(g2 * LANES, LANES)
                        d16 = dstb[b01][sl]
                        sv = plsc.load_gather(s_v, [d16])
                        a16 = e_v[sl] / (sv + 1e-16)
                        for j in range(LANES):
                            e = g2 * LANES + j
                            va = _bcast_lane(a16, lane, j)
                            for c16 in range(DSUB // LANES):
                                slc = pl.ds(c16 * LANES, LANES)
                                scb[b01][e, slc] = vrb[b01][e, slc] * va
                        return gcarry
                    lax.fori_loop(0, B5 // LANES, grp, 0)

                    pltpu.async_copy(scb[b01], slab.at[dstb[b01]], ssem,
                                     add=True)
                return carry
            lax.fori_loop(0, nblk // 2, pair, 0)

            pltpu.make_async_copy(vq[p].at[pl.ds(0, B5)], c0, ssem).wait()
            pltpu.make_async_copy(vq[p].at[pl.ds(0, B5)], c1, ssem).wait()

            plsc.subcore_barrier()
            out_row = p * NC + c
            for q0 in range(0, rows16, B5):
                pltpu.sync_copy(slab.at[pl.ds(s * rows16 + q0, B5)],
                                out_hbm.at[out_row,
                                           pl.ds(s * rows16 + q0, B5)])
            plsc.subcore_barrier()
    return k5


def _add_body(a_ref, b_ref, o_ref):
    o_ref[...] = a_ref[...] + b_ref[...]


def _pallas_add(a, b, block_rows=1024):
    r, d = a.shape
    grid = (r // block_rows,)
    return pl.pallas_call(
        _add_body,
        grid=grid,
        in_specs=[pl.BlockSpec((block_rows, d), lambda i: (i, 0)),
                  pl.BlockSpec((block_rows, d), lambda i: (i, 0))],
        out_specs=pl.BlockSpec((block_rows, d), lambda i: (i, 0)),
        out_shape=jax.ShapeDtypeStruct((r, d), jnp.float32),
    )(a, b)


def _round_up(x, m):
    return (x + m - 1) // m * m


def _sparse_attn_sc(q, k, v, src, dst, use_max):
    NdP = q.shape[0]
    EP = src.shape[0]
    qk = jnp.concatenate([q, k], axis=0)
    sd = jnp.concatenate([dst.reshape(-1, BE),
                          src.reshape(-1, BE) + NdP], axis=1)
    if use_max:
        l, mpart = _k1_max(EP, NdP)(qk, sd, dst)
        mfin = _k2_merge(NdP)(mpart)
        e, spart = _k3_expsum(EP, NdP, True)(dst, l, mfin)
    else:
        e, spart = _k1_fused(EP, NdP)(qk, sd, dst)
    dsub = 128 if NdP <= 12288 else 64
    npass = D // dsub
    vq = [v[:, i * dsub:(i + 1) * dsub] for i in range(npass)]
    srows = NdP // 128
    s = _pallas_add(spart[0].reshape(srows, 128),
                    spart[1].reshape(srows, 128),
                    block_rows=srows).reshape(NdP)
    outp = _k5_scatter(EP, NdP, dsub)(*vq, src, dst, e, s)
    a = jnp.transpose(outp[0::NC], (1, 0, 2)).reshape(NdP, D)
    b = jnp.transpose(outp[1::NC], (1, 0, 2)).reshape(NdP, D)
    return _pallas_add(a, b)


def _pad_edges(src, dst, ns, nd):
    E = src.shape[0]
    EP = _round_up(E, NW * B)
    src = jnp.pad(src.astype(jnp.int32), (0, EP - E), constant_values=ns)
    dst = jnp.pad(dst.astype(jnp.int32), (0, EP - E), constant_values=nd)
    return src, dst


def kernel(x_node, x_tri, params, node_edge_index, tri_edge_index,
           nt_edge_index, tn_edge_index):
    t, n, d = x_node.shape
    _, m, _ = x_tri.shape
    assert t == 1 and d == D
    NP = _round_up(n, 1024)
    MP = _round_up(m, 1024)
    scale = d ** (-0.5)

    xn = jnp.pad(x_node.reshape(n, d), ((0, NP - n), (0, 0)))
    xt = jnp.pad(x_tri.reshape(m, d), ((0, MP - m), (0, 0)))

    ne_src, ne_dst = _pad_edges(node_edge_index[0], node_edge_index[1], n, n)
    te_src, te_dst = _pad_edges(tri_edge_index[0], tri_edge_index[1], m, m)
    nt_src, nt_dst = _pad_edges(nt_edge_index[0], nt_edge_index[1], n, m)
    tn_src, tn_dst = _pad_edges(tn_edge_index[0], tn_edge_index[1], m, n)

    p = params['nsa']
    q, k, v = _qkv(xn, p, scale)
    h_node = _sparse_attn_sc(q, k, v, ne_src, ne_dst, False)

    p = params['n2t']
    q = _matmul_bias(xt, p['Wq'] * scale, p['bq'] * scale)
    kv = _matmul_bias(h_node,
                      jnp.concatenate([p['Wk'], p['Wv']], axis=1),
                      jnp.concatenate([p['bk'], p['bv']], axis=0))
    aggr = _sparse_attn_sc(q, kv[:, :D], kv[:, D:], nt_src, nt_dst, False)
    h_tri = _matmul_bias(aggr, p['Wo'], p['bo'], res=xt)

    p = params['tsa']
    q, k, v = _qkv(h_tri, p, d ** 0.5)
    h_tri2 = _sparse_attn_sc(q, k, v, te_src, te_dst, True)

    p = params['t2n']
    q = _matmul_bias(h_node, p['Wq'] * scale, p['bq'] * scale)
    kv = _matmul_bias(h_tri2,
                      jnp.concatenate([p['Wk'], p['Wv']], axis=1),
                      jnp.concatenate([p['bk'], p['bv']], axis=0))
    aggr = _sparse_attn_sc(q, kv[:, :D], kv[:, D:], tn_src, tn_dst, False)
    out_node = _matmul_bias(aggr, p['Wo'], p['bo'], res=h_node)
    return out_node[:n].reshape(t, n, d)
